# baseline (speedup 1.0000x reference)
import functools

import jax
import jax.numpy as jnp
from jax.experimental import pallas as pl
from jax.experimental.pallas import tpu as pltpu
from jax.experimental.pallas import tpu_sc as plsc

B, L, D = 1024, 200, 128
N = B * L
WIN = 128
EPS = 1e-7
BB = 64
G = 2


def _sc_gather(table, idx3, n):
    mesh = plsc.VectorSubcoreMesh(
        core_axis_name="c", subcore_axis_name="s", num_cores=1)

    width = table.shape[1]

    @functools.partial(
        pl.kernel,
        out_type=jax.ShapeDtypeStruct((n, width), table.dtype),
        mesh=mesh,
        scratch_types=[pltpu.SemaphoreType.DMA],
    )
    def k(table_hbm, idx_hbm, out_hbm, sem):
        def body(i_vmem, o_vmem):
            copies = [
                pltpu.async_copy(
                    table_hbm.at[i_vmem.at[0, g]],
                    o_vmem.at[pl.ds(g * WIN, WIN)],
                    sem,
                )
                for g in range(G)
            ]
            for c in copies:
                c.wait()

        pltpu.emit_pipeline(
            body,
            grid=(n // (G * WIN),),
            in_specs=[pl.BlockSpec((1, G, WIN), lambda i: (i, 0, 0))],
            out_specs=[pl.BlockSpec((G * WIN, width), lambda i: (i, 0))],
            core_axis_name=("c", "s"),
            dimension_semantics=(pltpu.PARALLEL,),
        )(idx_hbm, out_hbm)

    return k(table, idx3)


def _ln_math(x_ref, p_ref, g_ref, b_ref, o_ref):
    x = x_ref[...] + p_ref[...]
    m = jnp.mean(x, axis=-1, keepdims=True)
    c = x - m
    v = jnp.mean(c * c, axis=-1, keepdims=True)
    o_ref[...] = c * jax.lax.rsqrt(v + EPS) * g_ref[...] + b_ref[...]


def _ln_body_acc(x_ref, p_ref, g_ref, b_ref, acc_ref, o_ref):
    del acc_ref
    _ln_math(x_ref, p_ref, g_ref, b_ref, o_ref)


_DENSE_SPECS = [
    pl.BlockSpec((1, L, D), lambda i: (0, 0, 0)),
    pl.BlockSpec((1, 1, D), lambda i: (0, 0, 0)),
    pl.BlockSpec((1, 1, D), lambda i: (0, 0, 0)),
]


def _tc_add_ln_chunk(gath_c, pos, gamma, beta, acc, base_row, bc):
    base = base_row // BB
    x_spec = pl.BlockSpec((BB, L, D), lambda i: (i, 0, 0))
    out_spec = pl.BlockSpec((BB, L, D), lambda i: (base + i, 0, 0))
    common = dict(
        grid=(bc // BB,),
        out_specs=out_spec,
        out_shape=jax.ShapeDtypeStruct((B, L, D), jnp.float32),
        compiler_params=pltpu.CompilerParams(
            dimension_semantics=("arbitrary",),
        ),
    )
    if acc is None:
        return pl.pallas_call(
            _ln_math,
            in_specs=[x_spec] + _DENSE_SPECS,
            **common,
        )(gath_c, pos, gamma, beta)
    return pl.pallas_call(
        _ln_body_acc,
        in_specs=[x_spec] + _DENSE_SPECS + [pl.BlockSpec(memory_space=pl.ANY)],
        input_output_aliases={4: 0},
        **common,
    )(gath_c, pos, gamma, beta, acc)


CHUNKS = (256, 256, 256, 256)


def kernel(input_ids, word_embeddings, position_embeddings, ln_gamma, ln_beta):
    pos = position_embeddings[:L].reshape(1, L, D)
    g = ln_gamma.reshape(1, 1, D)
    b = ln_beta.reshape(1, 1, D)
    gaths = []
    off = 0
    for bc in CHUNKS:
        idx_c = input_ids[off:off + bc].reshape(bc * L // (G * WIN), G, WIN)
        gaths.append(_sc_gather(word_embeddings, idx_c, bc * L).reshape(bc, L, D))
        off += bc
    acc = None
    off = 0
    for gath_c, bc in zip(gaths, CHUNKS):
        acc = _tc_add_ln_chunk(gath_c, pos, g, b, acc, off, bc)
        off += bc
    return acc


# speedup vs baseline: 1.1993x; 1.1993x over previous
import functools

import jax
import jax.numpy as jnp
from jax.experimental import pallas as pl
from jax.experimental.pallas import tpu as pltpu
from jax.experimental.pallas import tpu_sc as plsc

B, L, D = 1024, 200, 128
N = B * L
WIN = 128
EPS = 1e-7
BB = 64
G = 2


def _sc_gather(table, idx3, n):
    mesh = plsc.VectorSubcoreMesh(core_axis_name="c", subcore_axis_name="s")

    width = table.shape[1]

    @functools.partial(
        pl.kernel,
        out_type=jax.ShapeDtypeStruct((n, width), table.dtype),
        mesh=mesh,
        scratch_types=[pltpu.SemaphoreType.DMA],
    )
    def k(table_hbm, idx_hbm, out_hbm, sem):
        def body(i_vmem, o_vmem):
            copies = [
                pltpu.async_copy(
                    table_hbm.at[i_vmem.at[0, g]],
                    o_vmem.at[pl.ds(g * WIN, WIN)],
                    sem,
                )
                for g in range(G)
            ]
            for c in copies:
                c.wait()

        pltpu.emit_pipeline(
            body,
            grid=(n // (G * WIN),),
            in_specs=[pl.BlockSpec((1, G, WIN), lambda i: (i, 0, 0))],
            out_specs=[pl.BlockSpec((G * WIN, width), lambda i: (i, 0))],
            core_axis_name=("c", "s"),
            dimension_semantics=(pltpu.PARALLEL,),
        )(idx_hbm, out_hbm)

    return k(table, idx3)


def _ln_math(x_ref, p_ref, g_ref, b_ref, o_ref):
    x = x_ref[...] + p_ref[...]
    m = jnp.mean(x, axis=-1, keepdims=True)
    c = x - m
    v = jnp.mean(c * c, axis=-1, keepdims=True)
    o_ref[...] = c * jax.lax.rsqrt(v + EPS) * g_ref[...] + b_ref[...]


def _ln_body_acc(x_ref, p_ref, g_ref, b_ref, acc_ref, o_ref):
    del acc_ref
    _ln_math(x_ref, p_ref, g_ref, b_ref, o_ref)


_DENSE_SPECS = [
    pl.BlockSpec((1, L, D), lambda i: (0, 0, 0)),
    pl.BlockSpec((1, 1, D), lambda i: (0, 0, 0)),
    pl.BlockSpec((1, 1, D), lambda i: (0, 0, 0)),
]


def _tc_add_ln_chunk(gath_c, pos, gamma, beta, acc, base_row, bc):
    base = base_row // BB
    x_spec = pl.BlockSpec((BB, L, D), lambda i: (i, 0, 0))
    out_spec = pl.BlockSpec((BB, L, D), lambda i: (base + i, 0, 0))
    common = dict(
        grid=(bc // BB,),
        out_specs=out_spec,
        out_shape=jax.ShapeDtypeStruct((B, L, D), jnp.float32),
        compiler_params=pltpu.CompilerParams(
            dimension_semantics=("arbitrary",),
        ),
    )
    if acc is None:
        return pl.pallas_call(
            _ln_math,
            in_specs=[x_spec] + _DENSE_SPECS,
            **common,
        )(gath_c, pos, gamma, beta)
    return pl.pallas_call(
        _ln_body_acc,
        in_specs=[x_spec] + _DENSE_SPECS + [pl.BlockSpec(memory_space=pl.ANY)],
        input_output_aliases={4: 0},
        **common,
    )(gath_c, pos, gamma, beta, acc)


CHUNKS = (512, 512)


def kernel(input_ids, word_embeddings, position_embeddings, ln_gamma, ln_beta):
    pos = position_embeddings[:L].reshape(1, L, D)
    g = ln_gamma.reshape(1, 1, D)
    b = ln_beta.reshape(1, 1, D)
    gaths = []
    off = 0
    for bc in CHUNKS:
        idx_c = input_ids[off:off + bc].reshape(bc * L // (G * WIN), G, WIN)
        gaths.append(_sc_gather(word_embeddings, idx_c, bc * L).reshape(bc, L, D))
        off += bc
    acc = None
    off = 0
    for gath_c, bc in zip(gaths, CHUNKS):
        acc = _tc_add_ln_chunk(gath_c, pos, g, b, acc, off, bc)
        off += bc
    return acc
